# Initial kernel scaffold; baseline (speedup 1.0000x reference)
import jax
import jax.numpy as jnp
from jax.experimental import pallas as pl


def kernel(batch_box_preds, batch_cls_preds):
    raise NotImplementedError("write your pallas kernel here")


# speedup vs baseline: 4.8507x; 4.8507x over previous
import jax
import jax.numpy as jnp
from jax.experimental import pallas as pl
from jax.experimental.pallas import tpu as pltpu

_NMS_POST = 256
_NMS_PRE = 1024
_NMS_THRESH = 0.7
_SCORE_THRESH = 0.1
_CHUNK = 8


def _nms_body(rows_ref, cols_ref, sco_ref, out_ref, valid_ref, mask_ref):
    x1r = rows_ref[0, 0:1, :]
    y1r = rows_ref[0, 1:2, :]
    x2r = rows_ref[0, 2:3, :]
    y2r = rows_ref[0, 3:4, :]
    area_r = jnp.maximum(x2r - x1r, 0.0) * jnp.maximum(y2r - y1r, 0.0)

    def mask_chunk(c, _):
        ch = cols_ref[0, pl.ds(c * _CHUNK, _CHUNK), :]
        x1c = ch[:, 0:1]
        y1c = ch[:, 1:2]
        x2c = ch[:, 2:3]
        y2c = ch[:, 3:4]
        area_c = jnp.maximum(x2c - x1c, 0.0) * jnp.maximum(y2c - y1c, 0.0)
        xx1 = jnp.maximum(x1c, x1r)
        yy1 = jnp.maximum(y1c, y1r)
        xx2 = jnp.minimum(x2c, x2r)
        yy2 = jnp.minimum(y2c, y2r)
        inter = jnp.maximum(xx2 - xx1, 0.0) * jnp.maximum(yy2 - yy1, 0.0)
        union = jnp.maximum(area_c + area_r - inter, 1e-8)
        iou = inter / union
        mask_ref[pl.ds(c * _CHUNK, _CHUNK), :] = (iou > _NMS_THRESH).astype(
            jnp.float32
        )
        return 0

    jax.lax.fori_loop(0, _NMS_PRE // _CHUNK, mask_chunk, 0)

    out_ref[...] = jnp.zeros((1, _NMS_POST, 8), jnp.float32)
    idx = jax.lax.broadcasted_iota(jnp.int32, (1, _NMS_PRE), 1)

    def greedy(i, carry):
        keep, cnt = carry
        row = mask_ref[pl.ds(i, 1), :]
        sup = jnp.max(keep * row)
        kept = jnp.logical_and(sup <= 0.0, sco_ref[0, 0, i] > _SCORE_THRESH)
        keep = jnp.where(jnp.logical_and(idx == i, kept), 1.0, keep)

        @pl.when(jnp.logical_and(kept, cnt < _NMS_POST))
        def _():
            out_ref[0, pl.ds(cnt, 1), :] = cols_ref[0, pl.ds(i, 1), :]

        return keep, cnt + kept.astype(jnp.int32)

    keep0 = jnp.zeros((1, _NMS_PRE), jnp.float32)
    _, cnt = jax.lax.fori_loop(0, _NMS_PRE, greedy, (keep0, jnp.int32(0)))
    valid_ref[0, 0, 0] = jnp.minimum(cnt, _NMS_POST)


def kernel(batch_box_preds, batch_cls_preds):
    b, _, _ = batch_box_preds.shape
    scores = jnp.max(batch_cls_preds, axis=-1)
    labels = jnp.argmax(batch_cls_preds, axis=-1).astype(jnp.float32)
    top_scores, top_idx = jax.lax.top_k(scores, _NMS_PRE)
    top_boxes = jnp.take_along_axis(batch_box_preds, top_idx[..., None], axis=1)
    top_labels = jnp.take_along_axis(labels, top_idx, axis=1)
    pad = jnp.zeros((b, _NMS_PRE, 2), jnp.float32)
    cols8 = jnp.concatenate(
        [top_boxes, top_scores[..., None], top_labels[..., None], pad], axis=-1
    )
    rows8 = jnp.transpose(cols8, (0, 2, 1))

    out, valid = pl.pallas_call(
        _nms_body,
        grid=(b,),
        in_specs=[
            pl.BlockSpec((1, 8, _NMS_PRE), lambda i: (i, 0, 0)),
            pl.BlockSpec((1, _NMS_PRE, 8), lambda i: (i, 0, 0)),
            pl.BlockSpec(
                (1, 1, _NMS_PRE), lambda i: (i, 0, 0), memory_space=pltpu.SMEM
            ),
        ],
        out_specs=[
            pl.BlockSpec((1, _NMS_POST, 8), lambda i: (i, 0, 0)),
            pl.BlockSpec((1, 1, 1), lambda i: (i, 0, 0), memory_space=pltpu.SMEM),
        ],
        out_shape=[
            jax.ShapeDtypeStruct((b, _NMS_POST, 8), jnp.float32),
            jax.ShapeDtypeStruct((b, 1, 1), jnp.int32),
        ],
        scratch_shapes=[pltpu.VMEM((_NMS_PRE, _NMS_PRE), jnp.float32)],
    )(rows8, cols8, top_scores[:, None, :])

    return out[:, :, :6], valid[:, 0, 0]


# speedup vs baseline: 9.1564x; 1.8877x over previous
import jax
import jax.numpy as jnp
from jax.experimental import pallas as pl
from jax.experimental.pallas import tpu as pltpu

_NMS_POST = 256
_NMS_PRE = 1024
_NMS_THRESH = 0.7
_SCORE_THRESH = 0.1
_CHUNK = 8


def _nms_body(rows_ref, cols_ref, sco_ref, out_ref, valid_ref, mask_ref):
    x1r = rows_ref[0, 0:1, :]
    y1r = rows_ref[0, 1:2, :]
    x2r = rows_ref[0, 2:3, :]
    y2r = rows_ref[0, 3:4, :]
    area_r = jnp.maximum(x2r - x1r, 0.0) * jnp.maximum(y2r - y1r, 0.0)

    def mask_chunk(c, _):
        ch = cols_ref[0, pl.ds(c * _CHUNK, _CHUNK), :]
        x1c = ch[:, 0:1]
        y1c = ch[:, 1:2]
        x2c = ch[:, 2:3]
        y2c = ch[:, 3:4]
        area_c = jnp.maximum(x2c - x1c, 0.0) * jnp.maximum(y2c - y1c, 0.0)
        xx1 = jnp.maximum(x1c, x1r)
        yy1 = jnp.maximum(y1c, y1r)
        xx2 = jnp.minimum(x2c, x2r)
        yy2 = jnp.minimum(y2c, y2r)
        inter = jnp.maximum(xx2 - xx1, 0.0) * jnp.maximum(yy2 - yy1, 0.0)
        union = jnp.maximum(area_c + area_r - inter, 1e-8)
        iou = inter / union
        mask_ref[pl.ds(c * _CHUNK, _CHUNK), :] = (iou > _NMS_THRESH).astype(
            jnp.float32
        )
        return 0

    jax.lax.fori_loop(0, _NMS_PRE // _CHUNK, mask_chunk, 0)

    out_ref[...] = jnp.zeros((1, _NMS_POST, 8), jnp.float32)
    idx = jax.lax.broadcasted_iota(jnp.int32, (1, _NMS_PRE), 1)

    def greedy_cond(carry):
        i, _, cnt = carry
        return jnp.logical_and(i < _NMS_PRE, cnt < _NMS_POST)

    def greedy(carry):
        i, keep, cnt = carry
        row = mask_ref[pl.ds(i, 1), :]
        sup = jnp.max(keep * row)
        kept = jnp.logical_and(sup <= 0.0, sco_ref[0, 0, i] > _SCORE_THRESH)
        keep = jnp.where(jnp.logical_and(idx == i, kept), 1.0, keep)

        @pl.when(kept)
        def _():
            out_ref[0, pl.ds(cnt, 1), :] = cols_ref[0, pl.ds(i, 1), :]

        return i + 1, keep, cnt + kept.astype(jnp.int32)

    keep0 = jnp.zeros((1, _NMS_PRE), jnp.float32)
    _, _, cnt = jax.lax.while_loop(
        greedy_cond, greedy, (jnp.int32(0), keep0, jnp.int32(0))
    )
    valid_ref[0, 0, 0] = jnp.minimum(cnt, _NMS_POST)


def kernel(batch_box_preds, batch_cls_preds):
    b, _, _ = batch_box_preds.shape
    scores = jnp.max(batch_cls_preds, axis=-1)
    labels = jnp.argmax(batch_cls_preds, axis=-1).astype(jnp.float32)
    top_scores, top_idx = jax.lax.top_k(scores, _NMS_PRE)
    top_boxes = jnp.take_along_axis(batch_box_preds, top_idx[..., None], axis=1)
    top_labels = jnp.take_along_axis(labels, top_idx, axis=1)
    pad = jnp.zeros((b, _NMS_PRE, 2), jnp.float32)
    cols8 = jnp.concatenate(
        [top_boxes, top_scores[..., None], top_labels[..., None], pad], axis=-1
    )
    rows8 = jnp.transpose(cols8, (0, 2, 1))

    out, valid = pl.pallas_call(
        _nms_body,
        grid=(b,),
        in_specs=[
            pl.BlockSpec((1, 8, _NMS_PRE), lambda i: (i, 0, 0)),
            pl.BlockSpec((1, _NMS_PRE, 8), lambda i: (i, 0, 0)),
            pl.BlockSpec(
                (1, 1, _NMS_PRE), lambda i: (i, 0, 0), memory_space=pltpu.SMEM
            ),
        ],
        out_specs=[
            pl.BlockSpec((1, _NMS_POST, 8), lambda i: (i, 0, 0)),
            pl.BlockSpec((1, 1, 1), lambda i: (i, 0, 0), memory_space=pltpu.SMEM),
        ],
        out_shape=[
            jax.ShapeDtypeStruct((b, _NMS_POST, 8), jnp.float32),
            jax.ShapeDtypeStruct((b, 1, 1), jnp.int32),
        ],
        scratch_shapes=[pltpu.VMEM((_NMS_PRE, _NMS_PRE), jnp.float32)],
    )(rows8, cols8, top_scores[:, None, :])

    return out[:, :, :6], valid[:, 0, 0]


# speedup vs baseline: 11.0715x; 1.2091x over previous
import jax
import jax.numpy as jnp
from jax.experimental import pallas as pl
from jax.experimental.pallas import tpu as pltpu

_NMS_POST = 256
_NMS_PRE = 1024
_NMS_THRESH = 0.7
_SCORE_THRESH = 0.1
_CHUNK = 8


def _nms_body(rows_ref, cols_ref, sco_ref, out_ref, valid_ref, mask_ref):
    x1r = rows_ref[0, 0:1, :]
    y1r = rows_ref[0, 1:2, :]
    x2r = rows_ref[0, 2:3, :]
    y2r = rows_ref[0, 3:4, :]
    area_r = jnp.maximum(x2r - x1r, 0.0) * jnp.maximum(y2r - y1r, 0.0)

    out_ref[...] = jnp.zeros((1, _NMS_POST, 8), jnp.float32)
    idx = jax.lax.broadcasted_iota(jnp.int32, (1, _NMS_PRE), 1)

    def outer_cond(carry):
        blk, _, cnt = carry
        return jnp.logical_and(blk < _NMS_PRE // _CHUNK, cnt < _NMS_POST)

    def outer(carry):
        blk, keep, cnt = carry
        r0 = blk * _CHUNK
        ch = cols_ref[0, pl.ds(r0, _CHUNK), :]
        x1c = ch[:, 0:1]
        y1c = ch[:, 1:2]
        x2c = ch[:, 2:3]
        y2c = ch[:, 3:4]
        area_c = jnp.maximum(x2c - x1c, 0.0) * jnp.maximum(y2c - y1c, 0.0)
        xx1 = jnp.maximum(x1c, x1r)
        yy1 = jnp.maximum(y1c, y1r)
        xx2 = jnp.minimum(x2c, x2r)
        yy2 = jnp.minimum(y2c, y2r)
        inter = jnp.maximum(xx2 - xx1, 0.0) * jnp.maximum(yy2 - yy1, 0.0)
        union = jnp.maximum(area_c + area_r - inter, 1e-8)
        iou = inter / union
        mask_ref[pl.ds(r0, _CHUNK), :] = (iou > _NMS_THRESH).astype(jnp.float32)

        def greedy(k, c2):
            keep, cnt = c2
            i = r0 + k
            row = mask_ref[pl.ds(i, 1), :]
            sup = jnp.max(keep * row)
            kept = jnp.logical_and(sup <= 0.0, sco_ref[0, 0, i] > _SCORE_THRESH)
            keep = jnp.where(jnp.logical_and(idx == i, kept), 1.0, keep)

            @pl.when(jnp.logical_and(kept, cnt < _NMS_POST))
            def _():
                out_ref[0, pl.ds(cnt, 1), :] = cols_ref[0, pl.ds(i, 1), :]

            return keep, cnt + kept.astype(jnp.int32)

        keep, cnt = jax.lax.fori_loop(0, _CHUNK, greedy, (keep, cnt))
        return blk + 1, keep, cnt

    keep0 = jnp.zeros((1, _NMS_PRE), jnp.float32)
    _, _, cnt = jax.lax.while_loop(
        outer_cond, outer, (jnp.int32(0), keep0, jnp.int32(0))
    )
    valid_ref[0, 0, 0] = jnp.minimum(cnt, _NMS_POST)


def kernel(batch_box_preds, batch_cls_preds):
    b, _, _ = batch_box_preds.shape
    scores = jnp.max(batch_cls_preds, axis=-1)
    labels = jnp.argmax(batch_cls_preds, axis=-1).astype(jnp.float32)
    top_scores, top_idx = jax.lax.top_k(scores, _NMS_PRE)
    top_boxes = jnp.take_along_axis(batch_box_preds, top_idx[..., None], axis=1)
    top_labels = jnp.take_along_axis(labels, top_idx, axis=1)
    pad = jnp.zeros((b, _NMS_PRE, 2), jnp.float32)
    cols8 = jnp.concatenate(
        [top_boxes, top_scores[..., None], top_labels[..., None], pad], axis=-1
    )
    rows8 = jnp.transpose(cols8, (0, 2, 1))

    out, valid = pl.pallas_call(
        _nms_body,
        grid=(b,),
        in_specs=[
            pl.BlockSpec((1, 8, _NMS_PRE), lambda i: (i, 0, 0)),
            pl.BlockSpec((1, _NMS_PRE, 8), lambda i: (i, 0, 0)),
            pl.BlockSpec(
                (1, 1, _NMS_PRE), lambda i: (i, 0, 0), memory_space=pltpu.SMEM
            ),
        ],
        out_specs=[
            pl.BlockSpec((1, _NMS_POST, 8), lambda i: (i, 0, 0)),
            pl.BlockSpec((1, 1, 1), lambda i: (i, 0, 0), memory_space=pltpu.SMEM),
        ],
        out_shape=[
            jax.ShapeDtypeStruct((b, _NMS_POST, 8), jnp.float32),
            jax.ShapeDtypeStruct((b, 1, 1), jnp.int32),
        ],
        scratch_shapes=[pltpu.VMEM((_NMS_PRE, _NMS_PRE), jnp.float32)],
    )(rows8, cols8, top_scores[:, None, :])

    return out[:, :, :6], valid[:, 0, 0]


# speedup vs baseline: 11.2384x; 1.0151x over previous
import jax
import jax.numpy as jnp
from jax.experimental import pallas as pl
from jax.experimental.pallas import tpu as pltpu

_NMS_POST = 256
_NMS_PRE = 1024
_NMS_THRESH = 0.7
_SCORE_THRESH = 0.1
_CHUNK = 8


def _nms_body(rows_ref, cols_ref, sco_ref, out_ref, valid_ref):
    x1r = rows_ref[0, 0:1, :]
    y1r = rows_ref[0, 1:2, :]
    x2r = rows_ref[0, 2:3, :]
    y2r = rows_ref[0, 3:4, :]
    area_r = jnp.maximum(x2r - x1r, 0.0) * jnp.maximum(y2r - y1r, 0.0)

    out_ref[...] = jnp.zeros((1, _NMS_POST, 8), jnp.float32)
    idx = jax.lax.broadcasted_iota(jnp.int32, (1, _NMS_PRE), 1)

    oh8 = jax.lax.broadcasted_iota(jnp.int32, (_CHUNK, 1), 0)

    def outer_cond(carry):
        blk, _, cnt = carry
        return jnp.logical_and(blk < _NMS_PRE // _CHUNK, cnt < _NMS_POST)

    def outer(carry):
        blk, keep, cnt = carry
        r0 = blk * _CHUNK
        ch = cols_ref[0, pl.ds(r0, _CHUNK), :]
        x1c = ch[:, 0:1]
        y1c = ch[:, 1:2]
        x2c = ch[:, 2:3]
        y2c = ch[:, 3:4]
        area_c = jnp.maximum(x2c - x1c, 0.0) * jnp.maximum(y2c - y1c, 0.0)
        xx1 = jnp.maximum(x1c, x1r)
        yy1 = jnp.maximum(y1c, y1r)
        xx2 = jnp.minimum(x2c, x2r)
        yy2 = jnp.minimum(y2c, y2r)
        inter = jnp.maximum(xx2 - xx1, 0.0) * jnp.maximum(yy2 - yy1, 0.0)
        union = jnp.maximum(area_c + area_r - inter, 1e-8)
        mask_blk = (inter / union > _NMS_THRESH).astype(jnp.float32)

        pre = jnp.max(mask_blk * keep, axis=1, keepdims=True)

        cht = ch.T
        xx1b = jnp.maximum(x1c, cht[0:1, :])
        yy1b = jnp.maximum(y1c, cht[1:2, :])
        xx2b = jnp.minimum(x2c, cht[2:3, :])
        yy2b = jnp.minimum(y2c, cht[3:4, :])
        interb = jnp.maximum(xx2b - xx1b, 0.0) * jnp.maximum(yy2b - yy1b, 0.0)
        unionb = jnp.maximum(area_c + area_c.T - interb, 1e-8)
        mask8 = (interb / unionb > _NMS_THRESH).astype(jnp.float32)

        kcol = jnp.zeros((_CHUNK, 1), jnp.float32)
        for k in range(_CHUNK):
            in_sup = jnp.max(kcol * mask8[:, k : k + 1])
            kept = jnp.logical_and(
                jnp.logical_and(pre[k, 0] <= 0.0, in_sup <= 0.0),
                sco_ref[0, 0, r0 + k] > _SCORE_THRESH,
            )
            kcol = jnp.where(jnp.logical_and(oh8 == k, kept), 1.0, kcol)
            keep = jnp.where(jnp.logical_and(idx == r0 + k, kept), 1.0, keep)

            @pl.when(jnp.logical_and(kept, cnt < _NMS_POST))
            def _(kept=kept, cnt=cnt, k=k):
                out_ref[0, pl.ds(cnt, 1), :] = cols_ref[0, pl.ds(r0 + k, 1), :]

            cnt = cnt + kept.astype(jnp.int32)
        return blk + 1, keep, cnt

    keep0 = jnp.zeros((1, _NMS_PRE), jnp.float32)
    _, _, cnt = jax.lax.while_loop(
        outer_cond, outer, (jnp.int32(0), keep0, jnp.int32(0))
    )
    valid_ref[0, 0, 0] = jnp.minimum(cnt, _NMS_POST)


def kernel(batch_box_preds, batch_cls_preds):
    b, _, _ = batch_box_preds.shape
    scores = jnp.max(batch_cls_preds, axis=-1)
    labels = jnp.argmax(batch_cls_preds, axis=-1).astype(jnp.float32)
    top_scores, top_idx = jax.lax.top_k(scores, _NMS_PRE)
    top_boxes = jnp.take_along_axis(batch_box_preds, top_idx[..., None], axis=1)
    top_labels = jnp.take_along_axis(labels, top_idx, axis=1)
    pad = jnp.zeros((b, _NMS_PRE, 2), jnp.float32)
    cols8 = jnp.concatenate(
        [top_boxes, top_scores[..., None], top_labels[..., None], pad], axis=-1
    )
    rows8 = jnp.transpose(cols8, (0, 2, 1))

    out, valid = pl.pallas_call(
        _nms_body,
        grid=(b,),
        in_specs=[
            pl.BlockSpec((1, 8, _NMS_PRE), lambda i: (i, 0, 0)),
            pl.BlockSpec((1, _NMS_PRE, 8), lambda i: (i, 0, 0)),
            pl.BlockSpec(
                (1, 1, _NMS_PRE), lambda i: (i, 0, 0), memory_space=pltpu.SMEM
            ),
        ],
        out_specs=[
            pl.BlockSpec((1, _NMS_POST, 8), lambda i: (i, 0, 0)),
            pl.BlockSpec((1, 1, 1), lambda i: (i, 0, 0), memory_space=pltpu.SMEM),
        ],
        out_shape=[
            jax.ShapeDtypeStruct((b, _NMS_POST, 8), jnp.float32),
            jax.ShapeDtypeStruct((b, 1, 1), jnp.int32),
        ],
    )(rows8, cols8, top_scores[:, None, :])

    return out[:, :, :6], valid[:, 0, 0]
